# baseline (speedup 1.0000x reference)
import functools

import jax
import jax.numpy as jnp
from jax.experimental import pallas as pl
from jax.experimental.pallas import tpu as pltpu

_N = 10000
_H = 8
_C = 128
_HC = 1024
_HID = 256
_STEPS = 12
_DT = 0.5
_MT = 400


def _mm_body(act, x_ref, w_ref, b_ref, o_ref):
    acc = jnp.dot(x_ref[...], w_ref[...], preferred_element_type=jnp.float32)
    acc = acc + b_ref[...]
    if act == "tanh":
        acc = jnp.tanh(acc)
    o_ref[...] = acc


def _matmul(x, w, b, act=None, nt=256):
    m, k = x.shape
    _, n = w.shape
    nt = min(nt, n)
    grid = (m // _MT, n // nt)
    return pl.pallas_call(
        functools.partial(_mm_body, act),
        grid=grid,
        in_specs=[
            pl.BlockSpec((_MT, k), lambda i, j: (i, 0)),
            pl.BlockSpec((k, nt), lambda i, j: (0, j)),
            pl.BlockSpec((1, nt), lambda i, j: (0, j)),
        ],
        out_specs=pl.BlockSpec((_MT, nt), lambda i, j: (i, j)),
        out_shape=jax.ShapeDtypeStruct((m, n), jnp.float32),
    )(x, w, b.reshape(1, -1))


def _bn_elu_body(h_ref, g_ref, b_ref, o_ref):
    h = h_ref[...]
    mu = jnp.mean(h, axis=0, keepdims=True)
    var = jnp.mean((h - mu) ** 2, axis=0, keepdims=True)
    y = (h - mu) / jnp.sqrt(var + 1e-5) * g_ref[...] + b_ref[...]
    o_ref[...] = jnp.where(y > 0, y, jnp.exp(jnp.minimum(y, 0.0)) - 1.0)


def _bn_elu(h, g, b, ct=128):
    n, c = h.shape
    return pl.pallas_call(
        _bn_elu_body,
        grid=(c // ct,),
        in_specs=[
            pl.BlockSpec((n, ct), lambda j: (0, j)),
            pl.BlockSpec((1, ct), lambda j: (0, j)),
            pl.BlockSpec((1, ct), lambda j: (0, j)),
        ],
        out_specs=pl.BlockSpec((n, ct), lambda j: (0, j)),
        out_shape=jax.ShapeDtypeStruct((n, c), jnp.float32),
    )(h, g.reshape(1, -1), b.reshape(1, -1))


def _diff_body(agg_ref, hd_ref, w1_ref, b1_ref, w2_ref, b2_ref, cl_ref, o_ref):
    z = jnp.dot(agg_ref[...], w1_ref[...], preferred_element_type=jnp.float32)
    z = _gelu(z + b1_ref[...])
    d = jnp.dot(z, w2_ref[...], preferred_element_type=jnp.float32)
    d = jnp.tanh(d + b2_ref[...])
    hd = hd_ref[...]
    cl = jnp.maximum(cl_ref[0, 0], 0.0)
    o_ref[...] = hd + (d - cl * hd) * _DT


def _diff_step(agg, hd, w1, b1, w2, b2, cl):
    return pl.pallas_call(
        _diff_body,
        grid=(_N // _MT,),
        in_specs=[
            pl.BlockSpec((_MT, _HID), lambda i: (i, 0)),
            pl.BlockSpec((_MT, _HID), lambda i: (i, 0)),
            pl.BlockSpec((_HID, 2 * _HID), lambda i: (0, 0)),
            pl.BlockSpec((1, 2 * _HID), lambda i: (0, 0)),
            pl.BlockSpec((2 * _HID, _HID), lambda i: (0, 0)),
            pl.BlockSpec((1, _HID), lambda i: (0, 0)),
            pl.BlockSpec((1, 1), lambda i: (0, 0)),
        ],
        out_specs=pl.BlockSpec((_MT, _HID), lambda i: (i, 0)),
        out_shape=jax.ShapeDtypeStruct((_N, _HID), jnp.float32),
    )(agg, hd, w1, b1.reshape(1, -1), w2, b2.reshape(1, -1), cl.reshape(1, 1))


def _gelu(x):
    return 0.5 * x * (1.0 + jax.lax.erf(x * 0.7071067811865476))


def _ln(x, g, b):
    mu = jnp.mean(x, axis=-1, keepdims=True)
    var = jnp.mean((x - mu) ** 2, axis=-1, keepdims=True)
    return (x - mu) / jnp.sqrt(var + 1e-5) * g + b


def _head_body(hd_ref, w0, b0, g0, l0, w1, b1, g1, l1, w2, b2, g2, l2, w3, b3,
               o_ref):
    y = hd_ref[...]
    y = _gelu(_ln(jnp.dot(y, w0[...], preferred_element_type=jnp.float32)
                  + b0[...], g0[...], l0[...]))
    y = _gelu(_ln(jnp.dot(y, w1[...], preferred_element_type=jnp.float32)
                  + b1[...], g1[...], l1[...]))
    y = _gelu(_ln(jnp.dot(y, w2[...], preferred_element_type=jnp.float32)
                  + b2[...], g2[...], l2[...]))
    o_ref[...] = jax.nn.sigmoid(
        jnp.dot(y, w3[...], preferred_element_type=jnp.float32) + b3[...])


def _head(hd, w0, b0, g0, l0, w1, b1, g1, l1, w2, b2, g2, l2, w3, b3):
    full = lambda r, c: pl.BlockSpec((r, c), lambda i: (0, 0))
    row = lambda c: pl.BlockSpec((1, c), lambda i: (0, 0))
    return pl.pallas_call(
        _head_body,
        grid=(_N // _MT,),
        in_specs=[
            pl.BlockSpec((_MT, _HID), lambda i: (i, 0)),
            full(_HID, 256), row(256), row(256), row(256),
            full(256, 128), row(128), row(128), row(128),
            full(128, 64), row(64), row(64), row(64),
            full(64, 1), row(1),
        ],
        out_specs=pl.BlockSpec((_MT, 1), lambda i: (i, 0)),
        out_shape=jax.ShapeDtypeStruct((_N, 1), jnp.float32),
    )(hd, w0, b0.reshape(1, -1), g0.reshape(1, -1), l0.reshape(1, -1),
      w1, b1.reshape(1, -1), g1.reshape(1, -1), l1.reshape(1, -1),
      w2, b2.reshape(1, -1), g2.reshape(1, -1), l2.reshape(1, -1),
      w3, b3.reshape(1, -1))


def _gat_edge(xl, xr, src, dst, att):
    n = xl.shape[0]
    xl3 = xl.reshape(n, _H, _C)
    xr3 = xr.reshape(n, _H, _C)
    msg = xl3[src]
    m = msg + xr3[dst]
    alpha = jnp.sum(jnp.where(m > 0, m, 0.2 * m) * att[None], axis=-1)
    amax = jax.ops.segment_max(alpha, dst, num_segments=n)
    ex = jnp.exp(alpha - amax[dst])
    denom = jax.ops.segment_sum(ex, dst, num_segments=n)
    a = ex / denom[dst]
    out = jax.ops.segment_sum(a[:, :, None] * msg, dst, num_segments=n)
    return out.reshape(n, _HC)


def _diffusion_agg(hd, src, dst, deg):
    agg = jax.ops.segment_sum(hd[src], dst, num_segments=_N) / deg
    return agg


def kernel(x, edge_index, g1_Wl, g1_bl, g1_Wr, g1_br, g1_att, g1_bias,
           g2_Wl, g2_bl, g2_Wr, g2_br, g2_att, g2_bias,
           bn1_g, bn1_b, bn2_g, bn2_b, proj_W, proj_b,
           d_W1, d_b1, d_W2, d_b2, clearance,
           h0_W, h0_b, h0_lg, h0_lb, h1_W, h1_b, h1_lg, h1_lb,
           h2_W, h2_b, h2_lg, h2_lb, h3_W, h3_b):
    src, dst = edge_index[0], edge_index[1]
    ar = jnp.arange(_N, dtype=edge_index.dtype)
    s2 = jnp.concatenate([src, ar])
    d2 = jnp.concatenate([dst, ar])

    xl1 = _matmul(x, g1_Wl, g1_bl)
    xr1 = _matmul(x, g1_Wr, g1_br)
    h1 = _gat_edge(xl1, xr1, s2, d2, g1_att)
    h1 = _bn_elu(h1, bn1_g, bn1_b)

    xl2 = _matmul(h1, g2_Wl, g2_bl)
    xr2 = _matmul(h1, g2_Wr, g2_br)
    h2 = _gat_edge(xl2, xr2, s2, d2, g2_att)
    h2 = _bn_elu(h2, bn2_g, bn2_b)

    hd = _matmul(h2, proj_W, proj_b, act="tanh")

    deg = jax.ops.segment_sum(jnp.ones(src.shape[0], dtype=jnp.float32), dst,
                              num_segments=_N)
    deg = jnp.maximum(deg, 1.0)[:, None]
    for _ in range(_STEPS):
        agg = _diffusion_agg(hd, src, dst, deg)
        hd = _diff_step(agg, hd, d_W1, d_b1, d_W2, d_b2, clearance)

    return _head(hd, h0_W, h0_b, h0_lg, h0_lb, h1_W, h1_b, h1_lg, h1_lb,
                 h2_W, h2_b, h2_lg, h2_lb, h3_W, h3_b)
